# baseline (speedup 1.0000x reference)
import functools

import jax
import jax.numpy as jnp
from jax import lax
from jax.experimental import pallas as pl
from jax.experimental.pallas import tpu as pltpu
from jax.experimental.pallas import tpu_sc as plsc

_NUM_CODONS = 64
_B, _L = 4096, 2048
_NC, _NS, _LANES = 2, 16, 16
_NW = _NC * _NS
_ROWS_PER_WORKER = _B // _NW
_RC = 16
_NCHUNK = _ROWS_PER_WORKER // _RC
_TBL = 80
_U = 16
_NA = 4


def _make_sc_kernel():
    mesh = plsc.VectorSubcoreMesh(core_axis_name="c", subcore_axis_name="s")

    @functools.partial(
        pl.kernel,
        mesh=mesh,
        out_type=jax.ShapeDtypeStruct((_B,), jnp.float32),
        compiler_params=pltpu.CompilerParams(needs_layout_passes=False),
        scratch_types=[
            pltpu.VMEM((_TBL * _LANES,), jnp.float32),
            pltpu.VMEM((_RC * _L,), jnp.int32),
            pltpu.VMEM((_RC * _L,), jnp.int32),
            pltpu.VMEM((_RC * 17,), jnp.float32),
            pltpu.VMEM((_RC * 17,), jnp.int32),
            pltpu.VMEM((_ROWS_PER_WORKER,), jnp.float32),
            pltpu.SemaphoreType.DMA,
            pltpu.SemaphoreType.DMA,
        ],
    )
    def sc_kernel(rna_hbm, table_hbm, out_hbm, table_v, buf0, buf1,
                  accmat, cntmat, out_v, sem0, sem1):
        wid = lax.axis_index("s") * _NC + lax.axis_index("c")
        base = wid * _ROWS_PER_WORKER

        pltpu.sync_copy(table_hbm, table_v)

        bufs = (buf0, buf1)
        sems = (sem0, sem1)
        iota = lax.iota(jnp.int32, _LANES)
        iota17 = iota * 17
        zf = jnp.zeros((_LANES,), jnp.float32)
        zi = jnp.zeros((_LANES,), jnp.int32)

        copies = [None, None]
        copies[0] = pltpu.async_copy(
            rna_hbm.at[pl.ds(base * _L, _RC * _L)], buf0, sem0)

        for c in range(_NCHUNK):
            cur = c % 2
            if c + 1 < _NCHUNK:
                copies[1 - cur] = pltpu.async_copy(
                    rna_hbm.at[pl.ds((base + (c + 1) * _RC) * _L, _RC * _L)],
                    bufs[1 - cur], sems[1 - cur])
            copies[cur].wait()
            buf = bufs[cur]

            def row_body(r, _, buf=buf):
                row_off = r * _L

                def body(s, carry, buf=buf, row_off=row_off):
                    accs, cnts, off = carry
                    accs, cnts = list(accs), list(cnts)
                    for u in range(_U):
                        v = buf[pl.ds(off + u * _LANES, _LANES)]
                        t = plsc.load_gather(table_v, [(v << 4) + iota])
                        a = u % _NA
                        accs[a] = accs[a] + t
                        cnts[a] = cnts[a] + ((v + 63) >> 6)
                    return tuple(accs), tuple(cnts), off + _U * _LANES

                accs, cnts, _ = lax.fori_loop(
                    0, _L // (_U * _LANES), body,
                    ((zf,) * _NA, (zi,) * _NA, row_off))
                acc = (accs[0] + accs[1]) + (accs[2] + accs[3])
                cnt = (cnts[0] + cnts[1]) + (cnts[2] + cnts[3])
                accmat[pl.ds(r * 17, _LANES)] = acc
                cntmat[pl.ds(r * 17, _LANES)] = cnt
                return 0

            lax.fori_loop(0, _RC, row_body, 0)

            att = zf
            ctt = zi
            for l in range(_LANES):
                att = att + plsc.load_gather(accmat, [iota17 + l])
                ctt = ctt + plsc.load_gather(cntmat, [iota17 + l])
            out_v[pl.ds(c * _RC, _RC)] = att / ctt.astype(jnp.float32)

        pltpu.sync_copy(out_v, out_hbm.at[pl.ds(base, _ROWS_PER_WORKER)])

    return sc_kernel


_SC_KERNEL = _make_sc_kernel()


def kernel(rna_data, tissue_id, W, b):
    del tissue_id
    table = jnp.zeros((_TBL,), jnp.float32).at[1:_NUM_CODONS + 1].set(W[0] + b[0])
    table_b = jnp.repeat(table, _LANES)
    y = _SC_KERNEL(rna_data.reshape(_B * _L), table_b)
    return y.reshape(_B, 1)


# speedup vs baseline: 1.0067x; 1.0067x over previous
import functools

import jax
import jax.numpy as jnp
from jax import lax
from jax.experimental import pallas as pl
from jax.experimental.pallas import tpu as pltpu
from jax.experimental.pallas import tpu_sc as plsc

_NUM_CODONS = 64
_B, _L = 4096, 2048
_NC, _NS, _LANES = 2, 16, 16
_NW = _NC * _NS
_ROWS_PER_WORKER = _B // _NW
_RC = 16
_NCHUNK = _ROWS_PER_WORKER // _RC
_TBL = 80
_U = 16
_NA = 4


def _make_sc_kernel():
    mesh = plsc.VectorSubcoreMesh(core_axis_name="c", subcore_axis_name="s")

    @functools.partial(
        pl.kernel,
        mesh=mesh,
        out_type=jax.ShapeDtypeStruct((_B,), jnp.float32),
        compiler_params=pltpu.CompilerParams(needs_layout_passes=False),
        scratch_types=[
            pltpu.VMEM((_TBL * _LANES,), jnp.float32),
            pltpu.VMEM((_RC * _L,), jnp.int32),
            pltpu.VMEM((_RC * _L,), jnp.int32),
            pltpu.VMEM((_RC * 17,), jnp.float32),
            pltpu.VMEM((_RC * 17,), jnp.int32),
            pltpu.VMEM((_ROWS_PER_WORKER,), jnp.float32),
            pltpu.SemaphoreType.DMA,
            pltpu.SemaphoreType.DMA,
        ],
    )
    def sc_kernel(rna_hbm, table_hbm, out_hbm, table_v, buf0, buf1,
                  accmat, cntmat, out_v, sem0, sem1):
        wid = lax.axis_index("s") * _NC + lax.axis_index("c")
        base = wid * _ROWS_PER_WORKER

        pltpu.sync_copy(table_hbm, table_v)

        bufs = (buf0, buf1)
        sems = (sem0, sem1)
        iota = lax.iota(jnp.int32, _LANES)
        iota17 = iota * 17
        zf = jnp.zeros((_LANES,), jnp.float32)
        zi = jnp.zeros((_LANES,), jnp.int32)

        copies = [None, None]
        copies[0] = pltpu.async_copy(
            rna_hbm.at[pl.ds(base * _L, _RC * _L)], buf0, sem0)

        for c in range(_NCHUNK):
            cur = c % 2
            if c + 1 < _NCHUNK:
                copies[1 - cur] = pltpu.async_copy(
                    rna_hbm.at[pl.ds((base + (c + 1) * _RC) * _L, _RC * _L)],
                    bufs[1 - cur], sems[1 - cur])
            copies[cur].wait()
            buf = bufs[cur]

            def row_body(r, _, buf=buf):
                row_off = r * _L

                def body(s, carry, buf=buf, row_off=row_off):
                    accs, cnts = carry
                    accs, cnts = list(accs), list(cnts)
                    off = row_off + s * (_NA * _LANES)
                    for u in range(_NA):
                        v = buf[pl.ds(off + u * _LANES, _LANES)]
                        t = plsc.load_gather(table_v, [(v << 4) + iota])
                        accs[u] = accs[u] + t
                        cnts[u] = cnts[u] + ((v + 63) >> 6)
                    return tuple(accs), tuple(cnts)

                accs, cnts = plsc.parallel_loop(
                    0, _L // (_NA * _LANES), 1, unroll=4,
                    carry=((zf,) * _NA, (zi,) * _NA))(body)
                acc = (accs[0] + accs[1]) + (accs[2] + accs[3])
                cnt = (cnts[0] + cnts[1]) + (cnts[2] + cnts[3])
                accmat[pl.ds(r * 17, _LANES)] = acc
                cntmat[pl.ds(r * 17, _LANES)] = cnt
                return 0

            lax.fori_loop(0, _RC, row_body, 0)

            att = zf
            ctt = zi
            for l in range(_LANES):
                att = att + plsc.load_gather(accmat, [iota17 + l])
                ctt = ctt + plsc.load_gather(cntmat, [iota17 + l])
            out_v[pl.ds(c * _RC, _RC)] = att / ctt.astype(jnp.float32)

        pltpu.sync_copy(out_v, out_hbm.at[pl.ds(base, _ROWS_PER_WORKER)])

    return sc_kernel


_SC_KERNEL = _make_sc_kernel()


def kernel(rna_data, tissue_id, W, b):
    del tissue_id
    table = jnp.zeros((_TBL,), jnp.float32).at[1:_NUM_CODONS + 1].set(W[0] + b[0])
    table_b = jnp.repeat(table, _LANES)
    y = _SC_KERNEL(rna_data.reshape(_B * _L), table_b)
    return y.reshape(_B, 1)


# speedup vs baseline: 1.0430x; 1.0360x over previous
import functools

import jax
import jax.numpy as jnp
from jax import lax
from jax.experimental import pallas as pl
from jax.experimental.pallas import tpu as pltpu
from jax.experimental.pallas import tpu_sc as plsc

_NUM_CODONS = 64
_B, _L = 4096, 2048
_NC, _NS, _LANES = 2, 16, 16
_NW = _NC * _NS
_ROWS_PER_WORKER = _B // _NW
_RC = 16
_NCHUNK = _ROWS_PER_WORKER // _RC
_TBL = 80
_U = 16
_NA = 4


def _make_sc_kernel():
    mesh = plsc.VectorSubcoreMesh(core_axis_name="c", subcore_axis_name="s")

    @functools.partial(
        pl.kernel,
        mesh=mesh,
        out_type=jax.ShapeDtypeStruct((_B,), jnp.float32),
        compiler_params=pltpu.CompilerParams(needs_layout_passes=False),
        scratch_types=[
            pltpu.VMEM((_TBL * _LANES,), jnp.float32),
            pltpu.VMEM((_RC * _L,), jnp.int32),
            pltpu.VMEM((_RC * _L,), jnp.int32),
            pltpu.VMEM((_RC * 17,), jnp.float32),
            pltpu.VMEM((_RC * 17,), jnp.int32),
            pltpu.VMEM((_ROWS_PER_WORKER,), jnp.float32),
            pltpu.SemaphoreType.DMA,
            pltpu.SemaphoreType.DMA,
        ],
    )
    def sc_kernel(rna_hbm, table_hbm, out_hbm, table_v, buf0, buf1,
                  accmat, cntmat, out_v, sem0, sem1):
        wid = lax.axis_index("s") * _NC + lax.axis_index("c")
        base = wid * _ROWS_PER_WORKER

        pltpu.sync_copy(table_hbm, table_v)

        bufs = (buf0, buf1)
        sems = (sem0, sem1)
        iota = lax.iota(jnp.int32, _LANES)
        iota17 = iota * 17
        zf = jnp.zeros((_LANES,), jnp.float32)
        zi = jnp.zeros((_LANES,), jnp.int32)

        for r in range(_RC):
            pltpu.async_copy(
                rna_hbm.at[pl.ds((base + r) * _L, _L)],
                buf0.at[pl.ds(r * _L, _L)], sem0)

        for c in range(_NCHUNK):
            cur = c % 2
            buf = bufs[cur]

            def row_body(r, _, buf=buf, sem=sems[cur], c=c, cur=cur):
                pltpu.make_async_copy(
                    rna_hbm.at[pl.ds(0, _L)],
                    buf.at[pl.ds(r * _L, _L)], sem).wait()
                if c + 1 < _NCHUNK:
                    pltpu.async_copy(
                        rna_hbm.at[pl.ds((base + (c + 1) * _RC) * _L + r * _L,
                                         _L)],
                        bufs[1 - cur].at[pl.ds(r * _L, _L)], sems[1 - cur])
                row_off = r * _L

                def body(s, carry, buf=buf, row_off=row_off):
                    accs, cnts = carry
                    accs, cnts = list(accs), list(cnts)
                    off = row_off + s * (_NA * _LANES)
                    for u in range(_NA):
                        v = buf[pl.ds(off + u * _LANES, _LANES)]
                        t = plsc.load_gather(table_v, [(v << 4) + iota])
                        accs[u] = accs[u] + t
                        cnts[u] = cnts[u] + ((v + 63) >> 6)
                    return tuple(accs), tuple(cnts)

                accs, cnts = plsc.parallel_loop(
                    0, _L // (_NA * _LANES), 1, unroll=4,
                    carry=((zf,) * _NA, (zi,) * _NA))(body)
                acc = (accs[0] + accs[1]) + (accs[2] + accs[3])
                cnt = (cnts[0] + cnts[1]) + (cnts[2] + cnts[3])
                accmat[pl.ds(r * 17, _LANES)] = acc
                cntmat[pl.ds(r * 17, _LANES)] = cnt
                return 0

            lax.fori_loop(0, _RC, row_body, 0)

            att = zf
            ctt = zi
            for l in range(_LANES):
                att = att + plsc.load_gather(accmat, [iota17 + l])
                ctt = ctt + plsc.load_gather(cntmat, [iota17 + l])
            out_v[pl.ds(c * _RC, _RC)] = att / ctt.astype(jnp.float32)

        pltpu.sync_copy(out_v, out_hbm.at[pl.ds(base, _ROWS_PER_WORKER)])

    return sc_kernel


_SC_KERNEL = _make_sc_kernel()


def kernel(rna_data, tissue_id, W, b):
    del tissue_id
    table = jnp.zeros((_TBL,), jnp.float32).at[1:_NUM_CODONS + 1].set(W[0] + b[0])
    table_b = jnp.repeat(table, _LANES)
    y = _SC_KERNEL(rna_data.reshape(_B * _L), table_b)
    return y.reshape(_B, 1)
